# baseline (speedup 1.0000x reference)
import functools

import jax
import jax.numpy as jnp
from jax import lax
from jax.experimental import pallas as pl
from jax.experimental.pallas import tpu as pltpu
from jax.experimental.pallas import tpu_sc as plsc

N = 10000
E = 320000
R = 8
D = 128
RT = R + 1

NC = 2
NS = 16
NW = NC * NS
EPT = E // NW
CH = 80
NCHUNK = EPT // CH
NPT = N // NS
CPB = 25
NBLK = EPT // (CPB * CH)
NROW = NW * NBLK
ACH = 40
ABLK = EPT // (CPB * ACH)
AROW = NW * ABLK

_mesh = functools.partial(
    plsc.VectorSubcoreMesh, core_axis_name="c", subcore_axis_name="s",
    num_cores=NC, num_subcores=NS)

_sc_params = pltpu.CompilerParams(needs_layout_passes=False)

_L = 16


def _sc_prep_body(ei_h, rel_h, zeros_h,
                  gidx_h, comb_h, parts_h,
                  src_v, dst_v, rel_v, gidx_v, comb_v, ones_v, sem, cnt_sp):
    c = lax.axis_index("c")
    s = lax.axis_index("s")
    wid = c * NS + s
    coff = jnp.minimum(s * 5120, N * R - 5120)
    pltpu.sync_copy(zeros_h.at[pl.ds(coff, 5120)], cnt_sp.at[pl.ds(coff, 5120)])

    def fill_ones(j, _):
        ones_v[pl.ds(j * _L, _L)] = jnp.ones((_L,), jnp.float32)
        return 0

    lax.fori_loop(0, CH // _L, fill_ones, 0)
    plsc.subcore_barrier()

    def block(bk, _):
        base = wid * NBLK + bk
        pltpu.sync_copy(ei_h.at[0, base], src_v)
        pltpu.sync_copy(ei_h.at[1, base], dst_v)
        pltpu.sync_copy(rel_h.at[base], rel_v)

        def lane(i, _):
            for m in range(CH // _L):
                sl = pl.ds(m * _L, _L)
                sv = src_v[i, sl]
                dv = dst_v[i, sl]
                rv = rel_v[i, sl]
                gidx_v[i, sl] = rv * N + sv
                comb_v[i, sl] = dv * R + rv
            return 0

        lax.fori_loop(0, CPB, lane, 0)
        pltpu.sync_copy(gidx_v, gidx_h.at[base])
        pltpu.sync_copy(comb_v, comb_h.at[base])
        for j in range(CPB):
            pltpu.async_copy(ones_v, cnt_sp.at[comb_v.at[j]], sem, add=True)
        for j in range(CPB):
            pltpu.make_async_copy(ones_v, cnt_sp.at[comb_v.at[j]], sem).wait()
        return 0

    lax.fori_loop(0, NBLK, block, 0)
    plsc.subcore_barrier()
    pltpu.sync_copy(cnt_sp.at[pl.ds(coff, 5120)],
                    parts_h.at[c, pl.ds(coff, 5120)])


def _sc_prep(ei, rel, zeros_nr):
    k = functools.partial(
        pl.kernel,
        out_type=[jax.ShapeDtypeStruct((NROW, CPB, CH), jnp.int32),
                  jax.ShapeDtypeStruct((NROW, CPB, CH), jnp.int32),
                  jax.ShapeDtypeStruct((NC, N * R), jnp.float32)],
        mesh=_mesh(),
        compiler_params=_sc_params,
        scratch_types=[
            pltpu.VMEM((CPB, CH), jnp.int32),
            pltpu.VMEM((CPB, CH), jnp.int32),
            pltpu.VMEM((CPB, CH), jnp.int32),
            pltpu.VMEM((CPB, CH), jnp.int32),
            pltpu.VMEM((CPB, CH), jnp.int32),
            pltpu.VMEM((CH,), jnp.float32),
            pltpu.SemaphoreType.DMA,
            pltpu.VMEM_SHARED((N * R,), jnp.float32),
        ],
    )(_sc_prep_body)
    return k(ei.reshape(2, NROW, CPB, CH), rel.reshape(NROW, CPB, CH),
             zeros_nr)


_CB = 3200


def _sc_norm_body(p0_h, p1_h, comb_h, norm_h, cnt_v, tmp_v, comb_v, norm_v):
    c = lax.axis_index("c")
    s = lax.axis_index("s")
    wid = c * NS + s
    pltpu.sync_copy(p0_h, cnt_v)

    def merge_blk(b, _):
        pltpu.sync_copy(p1_h.at[pl.ds(b * _CB, _CB)], tmp_v)

        def merge_lane(i, _):
            for u in range(4):
                sl = pl.ds(b * _CB + (i * 4 + u) * _L, _L)
                cnt_v[sl] = cnt_v[sl] + tmp_v[pl.ds((i * 4 + u) * _L, _L)]
            return 0

        lax.fori_loop(0, _CB // (_L * 4), merge_lane, 0)
        return 0

    lax.fori_loop(0, (N * R) // _CB, merge_blk, 0)

    def block(bk, _):
        base = wid * NBLK + bk
        pltpu.sync_copy(comb_h.at[base], comb_v)

        def lane(i, _):
            for m in range(CH // _L):
                sl = pl.ds(m * _L, _L)
                cv = plsc.load_gather(cnt_v, [comb_v[i, sl]])
                norm_v[i, sl] = 1.0 / jnp.maximum(cv, 1.0)
            return 0

        lax.fori_loop(0, CPB, lane, 0)
        pltpu.sync_copy(norm_v, norm_h.at[base])
        return 0

    lax.fori_loop(0, NBLK, block, 0)


def _sc_norm(parts, comb):
    k = functools.partial(
        pl.kernel,
        out_type=jax.ShapeDtypeStruct((NROW, CPB, CH), jnp.float32),
        mesh=_mesh(),
        compiler_params=_sc_params,
        scratch_types=[
            pltpu.VMEM((N * R,), jnp.float32),
            pltpu.VMEM((_CB,), jnp.float32),
            pltpu.VMEM((CPB, CH), jnp.int32),
            pltpu.VMEM((CPB, CH), jnp.float32),
        ],
    )(_sc_norm_body)
    return k(parts[0], parts[1], comb)


_RING = 5
_PREF = 2


def _sc_agg_body(xw_h, gidx_h, ei_h, norm_h, zeros_h, parts_h,
                 gidx_v, dst_v, norm_v,
                 r0, r1, r2, r3, r4,
                 g0, g1, g2, g3, g4,
                 s0, s1, s2, s3, s4,
                 acc_sp):
    rows = (r0, r1, r2, r3, r4)
    gsem = (g0, g1, g2, g3, g4)
    ssem = (s0, s1, s2, s3, s4)
    c = lax.axis_index("c")
    s = lax.axis_index("s")
    wid = c * NS + s
    roff = jnp.minimum(s * 640, N - 640)
    pltpu.sync_copy(zeros_h.at[pl.ds(roff, 640), :],
                    acc_sp.at[pl.ds(roff, 640), :])
    plsc.subcore_barrier()

    def issue_gather(t, b):
        pltpu.async_copy(xw_h.at[gidx_v.at[t]], rows[b], gsem[b])

    def wait_gather(t, b):
        pltpu.make_async_copy(xw_h.at[gidx_v.at[t]], rows[b], gsem[b]).wait()

    def issue_scatter(t, b):
        pltpu.async_copy(rows[b], acc_sp.at[dst_v.at[t]], ssem[b], add=True)

    def drain_scatter(b):
        pltpu.make_async_copy(rows[b], acc_sp.at[dst_v.at[0]], ssem[b]).wait()

    def block(bk, _):
        base = wid * ABLK + bk
        pltpu.sync_copy(gidx_h.at[base], gidx_v)
        pltpu.sync_copy(ei_h.at[1, base], dst_v)
        pltpu.sync_copy(norm_h.at[base], norm_v)
        issue_gather(0, 0)
        issue_gather(1, 1)

        def group(g, _):
            for b in range(_RING):
                t = g * _RING + b
                wait_gather(t, b)

                def row(j, _):
                    nv = plsc.load_gather(
                        norm_v, [jnp.full((_L,), t, jnp.int32),
                                 jnp.full((_L,), j, jnp.int32)])
                    for kk in range(D // _L):
                        sl = pl.ds(kk * _L, _L)
                        rows[b][j, sl] = rows[b][j, sl] * nv
                    return 0

                lax.fori_loop(0, ACH, row, 0)
                issue_scatter(t, b)
                b2 = (b + _PREF) % _RING
                if b + _PREF < _RING:
                    @pl.when(g >= 1)
                    def _():
                        drain_scatter(b2)
                    issue_gather(t + _PREF, b2)
                else:
                    @pl.when(g < (CPB // _RING) - 1)
                    def _():
                        drain_scatter(b2)
                        issue_gather(t + _PREF, b2)
            return 0

        lax.fori_loop(0, CPB // _RING, group, 0)
        for b in range(_RING):
            drain_scatter(b)
        return 0

    lax.fori_loop(0, ABLK, block, 0)
    plsc.subcore_barrier()
    pltpu.sync_copy(acc_sp.at[pl.ds(roff, 640), :],
                    parts_h.at[c, pl.ds(roff, 640), :])


def _sc_agg(xw, gidx, ei, norm, zeros_nd):
    k = functools.partial(
        pl.kernel,
        out_type=jax.ShapeDtypeStruct((NC, N, D), jnp.float32),
        mesh=_mesh(),
        compiler_params=_sc_params,
        scratch_types=(
            [pltpu.VMEM((CPB, ACH), jnp.int32),
             pltpu.VMEM((CPB, ACH), jnp.int32),
             pltpu.VMEM((CPB, ACH), jnp.float32)]
            + [pltpu.VMEM((ACH, D), jnp.float32)] * _RING
            + [pltpu.SemaphoreType.DMA] * (2 * _RING)
            + [pltpu.VMEM_SHARED((N, D), jnp.float32)]
        ),
    )(_sc_agg_body)
    return k(xw.reshape(N * RT, D), gidx.reshape(AROW, CPB, ACH),
             ei.reshape(2, AROW, CPB, ACH), norm.reshape(AROW, CPB, ACH),
             zeros_nd)


_BN = 400


def _einsum1_body(x_ref, w_ref, o_ref):
    xb = x_ref[...]
    for r in range(RT):
        o_ref[r] = jnp.dot(xb, w_ref[r], preferred_element_type=jnp.float32)


def _tc_einsum1(xb, Wall):
    return pl.pallas_call(
        _einsum1_body,
        grid=(N // _BN,),
        in_specs=[
            pl.BlockSpec((_BN, D), lambda i: (i, 0)),
            pl.BlockSpec((RT, D, D), lambda i: (0, 0, 0)),
        ],
        out_specs=pl.BlockSpec((RT, _BN, D), lambda i: (0, i, 0)),
        out_shape=jax.ShapeDtypeStruct((RT, N, D), jnp.float32),
    )(xb, Wall)


def _einsum2_body(p0_ref, p1_ref, xw_ref, b_ref, w_ref, o_ref):
    h = p0_ref[0] + p1_ref[0] + xw_ref[0] + b_ref[0]
    hb = jnp.maximum(h, 0.0).astype(jnp.bfloat16)
    for r in range(RT):
        o_ref[r] = jnp.dot(hb, w_ref[r], preferred_element_type=jnp.float32)


def _tc_einsum2(parts, xw1, b1, Wall):
    return pl.pallas_call(
        _einsum2_body,
        grid=(N // _BN,),
        in_specs=[
            pl.BlockSpec((1, _BN, D), lambda i: (0, i, 0)),
            pl.BlockSpec((1, _BN, D), lambda i: (1, i, 0)),
            pl.BlockSpec((1, _BN, D), lambda i: (R, i, 0)),
            pl.BlockSpec((1, D), lambda i: (0, 0)),
            pl.BlockSpec((RT, D, D), lambda i: (0, 0, 0)),
        ],
        out_specs=pl.BlockSpec((RT, _BN, D), lambda i: (0, i, 0)),
        out_shape=jax.ShapeDtypeStruct((RT, N, D), jnp.float32),
    )(parts, parts, xw1, b1, Wall)


def _combine_body(p0_ref, p1_ref, xw_ref, b_ref, o_ref):
    o_ref[...] = p0_ref[0] + p1_ref[0] + xw_ref[0] + b_ref[0]


def _tc_combine(parts, xw2, b2):
    return pl.pallas_call(
        _combine_body,
        grid=(N // _BN,),
        in_specs=[
            pl.BlockSpec((1, _BN, D), lambda i: (0, i, 0)),
            pl.BlockSpec((1, _BN, D), lambda i: (1, i, 0)),
            pl.BlockSpec((1, _BN, D), lambda i: (R, i, 0)),
            pl.BlockSpec((1, D), lambda i: (0, 0)),
        ],
        out_specs=pl.BlockSpec((_BN, D), lambda i: (i, 0)),
        out_shape=jax.ShapeDtypeStruct((N, D), jnp.float32),
    )(parts, parts, xw2, b2)


def kernel(x, edge_index, edge_type, W1, Wroot1, b1, W2, Wroot2, b2):
    rel = edge_type
    Wall1 = jnp.concatenate([W1, Wroot1[None]], axis=0).astype(jnp.bfloat16)
    Wall2 = jnp.concatenate([W2, Wroot2[None]], axis=0).astype(jnp.bfloat16)
    xb = x.astype(jnp.bfloat16)
    b1r = b1.reshape(1, D)
    b2r = b2.reshape(1, D)
    zeros_nr = jnp.zeros((N * R,), jnp.float32)
    zeros_nd = jnp.zeros((N, D), jnp.float32)

    gidx, comb, cnt_parts = _sc_prep(edge_index, rel, zeros_nr)
    norm = _sc_norm(cnt_parts, comb)

    xw1 = _tc_einsum1(xb, Wall1)
    parts1 = _sc_agg(xw1, gidx, edge_index, norm, zeros_nd)
    xw2 = _tc_einsum2(parts1, xw1, b1r, Wall2)
    parts2 = _sc_agg(xw2, gidx, edge_index, norm, zeros_nd)
    out = _tc_combine(parts2, xw2, b2r)
    return out


# speedup vs baseline: 1.0791x; 1.0791x over previous
import functools

import jax
import jax.numpy as jnp
from jax import lax
from jax.experimental import pallas as pl
from jax.experimental.pallas import tpu as pltpu
from jax.experimental.pallas import tpu_sc as plsc

N = 10000
E = 320000
R = 8
D = 128
RT = R + 1

NC = 2
NS = 16
NW = NC * NS
EPT = E // NW
CH = 80
NCHUNK = EPT // CH
NPT = N // NS
CPB = 25
NBLK = EPT // (CPB * CH)
NROW = NW * NBLK
ACH = 40
ABLK = EPT // (CPB * ACH)
AROW = NW * ABLK

_mesh = functools.partial(
    plsc.VectorSubcoreMesh, core_axis_name="c", subcore_axis_name="s",
    num_cores=NC, num_subcores=NS)

_sc_params = pltpu.CompilerParams(needs_layout_passes=False)

_L = 16


EPS = E // NS
BPS = NBLK * NC


def _sc_prep_body(ei_h, rel_h, zeros_h,
                  gidx_h, norm_h,
                  src_v, dst_v, rel_v, gidx_v, comb_v, ones_v, norm_v,
                  cnt_v, sem, cnt_sp):
    c = lax.axis_index("c")
    s = lax.axis_index("s")
    wid = c * NS + s
    coff = jnp.minimum(s * 5120, N * R - 5120)
    pltpu.sync_copy(zeros_h.at[pl.ds(coff, 5120)], cnt_sp.at[pl.ds(coff, 5120)])

    def fill_ones(j, _):
        ones_v[pl.ds(j * _L, _L)] = jnp.ones((_L,), jnp.float32)
        return 0

    lax.fori_loop(0, CH // _L, fill_ones, 0)
    plsc.subcore_barrier()

    def blockA(bk, _):
        base = s * BPS + bk
        pltpu.sync_copy(ei_h.at[0, base], src_v)
        pltpu.sync_copy(ei_h.at[1, base], dst_v)
        pltpu.sync_copy(rel_h.at[base], rel_v)

        def lane(i, _):
            for m in range(CH // _L):
                sl = pl.ds(m * _L, _L)
                rv = rel_v[i, sl]
                gidx_v[i, sl] = rv * N + src_v[i, sl]
                comb_v[i, sl] = dst_v[i, sl] * R + rv
            return 0

        lax.fori_loop(0, CPB, lane, 0)

        @pl.when(c == 0)
        def _():
            pltpu.sync_copy(gidx_v, gidx_h.at[base])

        for j in range(CPB):
            pltpu.async_copy(ones_v, cnt_sp.at[comb_v.at[j]], sem, add=True)
        for j in range(CPB):
            pltpu.make_async_copy(ones_v, cnt_sp.at[comb_v.at[j]], sem).wait()
        return 0

    lax.fori_loop(0, BPS, blockA, 0)
    plsc.subcore_barrier()
    pltpu.sync_copy(cnt_sp, cnt_v)

    def blockB(bk, _):
        base = wid * NBLK + bk
        pltpu.sync_copy(ei_h.at[1, base], dst_v)
        pltpu.sync_copy(rel_h.at[base], rel_v)

        def lane(i, _):
            for m in range(CH // _L):
                sl = pl.ds(m * _L, _L)
                cb = dst_v[i, sl] * R + rel_v[i, sl]
                cv = plsc.load_gather(cnt_v, [cb])
                norm_v[i, sl] = 1.0 / jnp.maximum(cv, 1.0)
            return 0

        lax.fori_loop(0, CPB, lane, 0)
        pltpu.sync_copy(norm_v, norm_h.at[base])
        return 0

    lax.fori_loop(0, NBLK, blockB, 0)


def _sc_prep(ei, rel, zeros_nr):
    k = functools.partial(
        pl.kernel,
        out_type=[jax.ShapeDtypeStruct((NROW, CPB, CH), jnp.int32),
                  jax.ShapeDtypeStruct((NROW, CPB, CH), jnp.float32)],
        mesh=_mesh(),
        compiler_params=_sc_params,
        scratch_types=[
            pltpu.VMEM((CPB, CH), jnp.int32),
            pltpu.VMEM((CPB, CH), jnp.int32),
            pltpu.VMEM((CPB, CH), jnp.int32),
            pltpu.VMEM((CPB, CH), jnp.int32),
            pltpu.VMEM((CPB, CH), jnp.int32),
            pltpu.VMEM((CH,), jnp.float32),
            pltpu.VMEM((CPB, CH), jnp.float32),
            pltpu.VMEM((N * R,), jnp.float32),
            pltpu.SemaphoreType.DMA,
            pltpu.VMEM_SHARED((N * R,), jnp.float32),
        ],
    )(_sc_prep_body)
    return k(ei.reshape(2, NROW, CPB, CH), rel.reshape(NROW, CPB, CH),
             zeros_nr)


_RING = 5
_PREF = 2


def _sc_agg_body(xw_h, gidx_h, ei_h, norm_h, zeros_h, parts_h,
                 gidx_v, dst_v, norm_v,
                 r0, r1, r2, r3, r4,
                 g0, g1, g2, g3, g4,
                 s0, s1, s2, s3, s4,
                 acc_sp):
    rows = (r0, r1, r2, r3, r4)
    gsem = (g0, g1, g2, g3, g4)
    ssem = (s0, s1, s2, s3, s4)
    c = lax.axis_index("c")
    s = lax.axis_index("s")
    wid = c * NS + s
    roff = jnp.minimum(s * 640, N - 640)
    pltpu.sync_copy(zeros_h.at[pl.ds(roff, 640), :],
                    acc_sp.at[pl.ds(roff, 640), :])
    plsc.subcore_barrier()

    def issue_gather(t, b):
        pltpu.async_copy(xw_h.at[gidx_v.at[t]], rows[b], gsem[b])

    def wait_gather(t, b):
        pltpu.make_async_copy(xw_h.at[gidx_v.at[t]], rows[b], gsem[b]).wait()

    def issue_scatter(t, b):
        pltpu.async_copy(rows[b], acc_sp.at[dst_v.at[t]], ssem[b], add=True)

    def drain_scatter(b):
        pltpu.make_async_copy(rows[b], acc_sp.at[dst_v.at[0]], ssem[b]).wait()

    def block(bk, _):
        base = wid * ABLK + bk
        pltpu.sync_copy(gidx_h.at[base], gidx_v)
        pltpu.sync_copy(ei_h.at[1, base], dst_v)
        pltpu.sync_copy(norm_h.at[base], norm_v)
        issue_gather(0, 0)
        issue_gather(1, 1)

        def group(g, _):
            for b in range(_RING):
                t = g * _RING + b
                wait_gather(t, b)

                def row(j, _):
                    nv = plsc.load_gather(
                        norm_v, [jnp.full((_L,), t, jnp.int32),
                                 jnp.full((_L,), j, jnp.int32)])
                    for kk in range(D // _L):
                        sl = pl.ds(kk * _L, _L)
                        rows[b][j, sl] = rows[b][j, sl] * nv
                    return 0

                lax.fori_loop(0, ACH, row, 0)
                issue_scatter(t, b)
                b2 = (b + _PREF) % _RING
                if b + _PREF < _RING:
                    @pl.when(g >= 1)
                    def _():
                        drain_scatter(b2)
                    issue_gather(t + _PREF, b2)
                else:
                    @pl.when(g < (CPB // _RING) - 1)
                    def _():
                        drain_scatter(b2)
                        issue_gather(t + _PREF, b2)
            return 0

        lax.fori_loop(0, CPB // _RING, group, 0)
        for b in range(_RING):
            drain_scatter(b)
        return 0

    lax.fori_loop(0, ABLK, block, 0)
    plsc.subcore_barrier()
    pltpu.sync_copy(acc_sp.at[pl.ds(roff, 640), :],
                    parts_h.at[c, pl.ds(roff, 640), :])


def _sc_agg(xw, gidx, ei, norm, zeros_nd):
    k = functools.partial(
        pl.kernel,
        out_type=jax.ShapeDtypeStruct((NC, N, D), jnp.float32),
        mesh=_mesh(),
        compiler_params=_sc_params,
        scratch_types=(
            [pltpu.VMEM((CPB, ACH), jnp.int32),
             pltpu.VMEM((CPB, ACH), jnp.int32),
             pltpu.VMEM((CPB, ACH), jnp.float32)]
            + [pltpu.VMEM((ACH, D), jnp.float32)] * _RING
            + [pltpu.SemaphoreType.DMA] * (2 * _RING)
            + [pltpu.VMEM_SHARED((N, D), jnp.float32)]
        ),
    )(_sc_agg_body)
    return k(xw.reshape(N * RT, D), gidx.reshape(AROW, CPB, ACH),
             ei.reshape(2, AROW, CPB, ACH), norm.reshape(AROW, CPB, ACH),
             zeros_nd)


_BN = 400


def _einsum1_body(x_ref, w_ref, o_ref):
    xb = x_ref[...]
    for r in range(RT):
        o_ref[r] = jnp.dot(xb, w_ref[r], preferred_element_type=jnp.float32)


def _tc_einsum1(xb, Wall):
    return pl.pallas_call(
        _einsum1_body,
        grid=(N // _BN,),
        in_specs=[
            pl.BlockSpec((_BN, D), lambda i: (i, 0)),
            pl.BlockSpec((RT, D, D), lambda i: (0, 0, 0)),
        ],
        out_specs=pl.BlockSpec((RT, _BN, D), lambda i: (0, i, 0)),
        out_shape=jax.ShapeDtypeStruct((RT, N, D), jnp.float32),
    )(xb, Wall)


def _einsum2_body(p0_ref, p1_ref, xw_ref, b_ref, w_ref, o_ref):
    h = p0_ref[0] + p1_ref[0] + xw_ref[0] + b_ref[0]
    hb = jnp.maximum(h, 0.0).astype(jnp.bfloat16)
    for r in range(RT):
        o_ref[r] = jnp.dot(hb, w_ref[r], preferred_element_type=jnp.float32)


def _tc_einsum2(parts, xw1, b1, Wall):
    return pl.pallas_call(
        _einsum2_body,
        grid=(N // _BN,),
        in_specs=[
            pl.BlockSpec((1, _BN, D), lambda i: (0, i, 0)),
            pl.BlockSpec((1, _BN, D), lambda i: (1, i, 0)),
            pl.BlockSpec((1, _BN, D), lambda i: (R, i, 0)),
            pl.BlockSpec((1, D), lambda i: (0, 0)),
            pl.BlockSpec((RT, D, D), lambda i: (0, 0, 0)),
        ],
        out_specs=pl.BlockSpec((RT, _BN, D), lambda i: (0, i, 0)),
        out_shape=jax.ShapeDtypeStruct((RT, N, D), jnp.float32),
    )(parts, parts, xw1, b1, Wall)


def _combine_body(p0_ref, p1_ref, xw_ref, b_ref, o_ref):
    o_ref[...] = p0_ref[0] + p1_ref[0] + xw_ref[0] + b_ref[0]


def _tc_combine(parts, xw2, b2):
    return pl.pallas_call(
        _combine_body,
        grid=(N // _BN,),
        in_specs=[
            pl.BlockSpec((1, _BN, D), lambda i: (0, i, 0)),
            pl.BlockSpec((1, _BN, D), lambda i: (1, i, 0)),
            pl.BlockSpec((1, _BN, D), lambda i: (R, i, 0)),
            pl.BlockSpec((1, D), lambda i: (0, 0)),
        ],
        out_specs=pl.BlockSpec((_BN, D), lambda i: (i, 0)),
        out_shape=jax.ShapeDtypeStruct((N, D), jnp.float32),
    )(parts, parts, xw2, b2)


def kernel(x, edge_index, edge_type, W1, Wroot1, b1, W2, Wroot2, b2):
    rel = edge_type
    Wall1 = jnp.concatenate([W1, Wroot1[None]], axis=0).astype(jnp.bfloat16)
    Wall2 = jnp.concatenate([W2, Wroot2[None]], axis=0).astype(jnp.bfloat16)
    xb = x.astype(jnp.bfloat16)
    b1r = b1.reshape(1, D)
    b2r = b2.reshape(1, D)
    zeros_nr = jnp.zeros((N * R,), jnp.float32)
    zeros_nd = jnp.zeros((N, D), jnp.float32)

    gidx, norm = _sc_prep(edge_index, rel, zeros_nr)

    xw1 = _tc_einsum1(xb, Wall1)
    parts1 = _sc_agg(xw1, gidx, edge_index, norm, zeros_nd)
    xw2 = _tc_einsum2(parts1, xw1, b1r, Wall2)
    parts2 = _sc_agg(xw2, gidx, edge_index, norm, zeros_nd)
    out = _tc_combine(parts2, xw2, b2r)
    return out


# speedup vs baseline: 1.1548x; 1.0701x over previous
import functools

import jax
import jax.numpy as jnp
from jax import lax
from jax.experimental import pallas as pl
from jax.experimental.pallas import tpu as pltpu
from jax.experimental.pallas import tpu_sc as plsc

N = 10000
E = 320000
R = 8
D = 128
RT = R + 1

NC = 2
NS = 16
NW = NC * NS
EPT = E // NW
CH = 80
NCHUNK = EPT // CH
NPT = N // NS
CPB = 25
NBLK = EPT // (CPB * CH)
NROW = NW * NBLK
ACH = 50
ABLK = EPT // (CPB * ACH)
AROW = NW * ABLK

_mesh = functools.partial(
    plsc.VectorSubcoreMesh, core_axis_name="c", subcore_axis_name="s",
    num_cores=NC, num_subcores=NS)

_sc_params = pltpu.CompilerParams(needs_layout_passes=False)

_L = 16


EPS = E // NS
BPS = NBLK * NC


def _sc_prep_body(ei_h, rel_h, zeros_h,
                  gidx_h, norm_h,
                  src_v, dst_v, rel_v, gidx_v, comb_v, ones_v, norm_v,
                  cnt_v, sem, cnt_sp):
    c = lax.axis_index("c")
    s = lax.axis_index("s")
    wid = c * NS + s
    coff = jnp.minimum(s * 5120, N * R - 5120)
    pltpu.sync_copy(zeros_h.at[pl.ds(coff, 5120)], cnt_sp.at[pl.ds(coff, 5120)])

    def fill_ones(j, _):
        ones_v[pl.ds(j * _L, _L)] = jnp.ones((_L,), jnp.float32)
        return 0

    lax.fori_loop(0, CH // _L, fill_ones, 0)
    plsc.subcore_barrier()

    def blockA(bk, _):
        base = s * BPS + bk
        pltpu.sync_copy(ei_h.at[0, base], src_v)
        pltpu.sync_copy(ei_h.at[1, base], dst_v)
        pltpu.sync_copy(rel_h.at[base], rel_v)

        def lane(i, _):
            for m in range(CH // _L):
                sl = pl.ds(m * _L, _L)
                rv = rel_v[i, sl]
                gidx_v[i, sl] = rv * N + src_v[i, sl]
                comb_v[i, sl] = dst_v[i, sl] * R + rv
            return 0

        lax.fori_loop(0, CPB, lane, 0)

        @pl.when(c == 0)
        def _():
            pltpu.sync_copy(gidx_v, gidx_h.at[base])

        for j in range(CPB):
            pltpu.async_copy(ones_v, cnt_sp.at[comb_v.at[j]], sem, add=True)
        for j in range(CPB):
            pltpu.make_async_copy(ones_v, cnt_sp.at[comb_v.at[j]], sem).wait()
        return 0

    lax.fori_loop(0, BPS, blockA, 0)
    plsc.subcore_barrier()
    pltpu.sync_copy(cnt_sp, cnt_v)

    def blockB(bk, _):
        base = wid * NBLK + bk
        pltpu.sync_copy(ei_h.at[1, base], dst_v)
        pltpu.sync_copy(rel_h.at[base], rel_v)

        def lane(i, _):
            for m in range(CH // _L):
                sl = pl.ds(m * _L, _L)
                cb = dst_v[i, sl] * R + rel_v[i, sl]
                cv = plsc.load_gather(cnt_v, [cb])
                norm_v[i, sl] = 1.0 / jnp.maximum(cv, 1.0)
            return 0

        lax.fori_loop(0, CPB, lane, 0)
        pltpu.sync_copy(norm_v, norm_h.at[base])
        return 0

    lax.fori_loop(0, NBLK, blockB, 0)


def _sc_prep(ei, rel, zeros_nr):
    k = functools.partial(
        pl.kernel,
        out_type=[jax.ShapeDtypeStruct((NROW, CPB, CH), jnp.int32),
                  jax.ShapeDtypeStruct((NROW, CPB, CH), jnp.float32)],
        mesh=_mesh(),
        compiler_params=_sc_params,
        scratch_types=[
            pltpu.VMEM((CPB, CH), jnp.int32),
            pltpu.VMEM((CPB, CH), jnp.int32),
            pltpu.VMEM((CPB, CH), jnp.int32),
            pltpu.VMEM((CPB, CH), jnp.int32),
            pltpu.VMEM((CPB, CH), jnp.int32),
            pltpu.VMEM((CH,), jnp.float32),
            pltpu.VMEM((CPB, CH), jnp.float32),
            pltpu.VMEM((N * R,), jnp.float32),
            pltpu.SemaphoreType.DMA,
            pltpu.VMEM_SHARED((N * R,), jnp.float32),
        ],
    )(_sc_prep_body)
    return k(ei.reshape(2, NROW, CPB, CH), rel.reshape(NROW, CPB, CH),
             zeros_nr)


_RING = 5
_PREF = 2


def _sc_agg_body(xw_h, gidx_h, ei_h, norm_h, zeros_h, parts_h,
                 gidx_v, dst_v, norm_v,
                 r0, r1, r2, r3, r4,
                 g0, g1, g2, g3, g4,
                 s0, s1, s2, s3, s4,
                 acc_sp):
    rows = (r0, r1, r2, r3, r4)
    gsem = (g0, g1, g2, g3, g4)
    ssem = (s0, s1, s2, s3, s4)
    c = lax.axis_index("c")
    s = lax.axis_index("s")
    wid = c * NS + s
    roff = jnp.minimum(s * 640, N - 640)
    pltpu.sync_copy(zeros_h.at[pl.ds(roff, 640), :],
                    acc_sp.at[pl.ds(roff, 640), :])
    plsc.subcore_barrier()

    def issue_gather(t, b):
        pltpu.async_copy(xw_h.at[gidx_v.at[t]], rows[b], gsem[b])

    def wait_gather(t, b):
        pltpu.make_async_copy(xw_h.at[gidx_v.at[t]], rows[b], gsem[b]).wait()

    def issue_scatter(t, b):
        pltpu.async_copy(rows[b], acc_sp.at[dst_v.at[t]], ssem[b], add=True)

    def drain_scatter(b):
        pltpu.make_async_copy(rows[b], acc_sp.at[dst_v.at[0]], ssem[b]).wait()

    def block(bk, _):
        base = wid * ABLK + bk
        pltpu.sync_copy(gidx_h.at[base], gidx_v)
        pltpu.sync_copy(ei_h.at[1, base], dst_v)
        pltpu.sync_copy(norm_h.at[base], norm_v)
        issue_gather(0, 0)
        issue_gather(1, 1)

        def group(g, _):
            for b in range(_RING):
                t = g * _RING + b
                wait_gather(t, b)

                def row(j, _):
                    nv = plsc.load_gather(
                        norm_v, [jnp.full((_L,), t, jnp.int32),
                                 jnp.full((_L,), j, jnp.int32)])
                    for kk in range(D // _L):
                        sl = pl.ds(kk * _L, _L)
                        rows[b][j, sl] = rows[b][j, sl] * nv
                    return 0

                lax.fori_loop(0, ACH, row, 0)
                issue_scatter(t, b)
                b2 = (b + _PREF) % _RING
                if b + _PREF < _RING:
                    @pl.when(g >= 1)
                    def _():
                        drain_scatter(b2)
                    issue_gather(t + _PREF, b2)
                else:
                    @pl.when(g < (CPB // _RING) - 1)
                    def _():
                        drain_scatter(b2)
                        issue_gather(t + _PREF, b2)
            return 0

        lax.fori_loop(0, CPB // _RING, group, 0)
        for b in range(_RING):
            drain_scatter(b)
        return 0

    lax.fori_loop(0, ABLK, block, 0)
    plsc.subcore_barrier()
    pltpu.sync_copy(acc_sp.at[pl.ds(roff, 640), :],
                    parts_h.at[c, pl.ds(roff, 640), :])


def _sc_agg(xw, gidx, ei, norm, zeros_nd):
    k = functools.partial(
        pl.kernel,
        out_type=jax.ShapeDtypeStruct((NC, N, D), jnp.float32),
        mesh=_mesh(),
        compiler_params=_sc_params,
        scratch_types=(
            [pltpu.VMEM((CPB, ACH), jnp.int32),
             pltpu.VMEM((CPB, ACH), jnp.int32),
             pltpu.VMEM((CPB, ACH), jnp.float32)]
            + [pltpu.VMEM((ACH, D), jnp.float32)] * _RING
            + [pltpu.SemaphoreType.DMA] * (2 * _RING)
            + [pltpu.VMEM_SHARED((N, D), jnp.float32)]
        ),
    )(_sc_agg_body)
    return k(xw.reshape(N * RT, D), gidx.reshape(AROW, CPB, ACH),
             ei.reshape(2, AROW, CPB, ACH), norm.reshape(AROW, CPB, ACH),
             zeros_nd)


_BN = 400


def _einsum1_body(x_ref, w_ref, o_ref):
    xb = x_ref[...]
    for r in range(RT):
        o_ref[r] = jnp.dot(xb, w_ref[r], preferred_element_type=jnp.float32)


def _tc_einsum1(xb, Wall):
    return pl.pallas_call(
        _einsum1_body,
        grid=(N // _BN,),
        in_specs=[
            pl.BlockSpec((_BN, D), lambda i: (i, 0)),
            pl.BlockSpec((RT, D, D), lambda i: (0, 0, 0)),
        ],
        out_specs=pl.BlockSpec((RT, _BN, D), lambda i: (0, i, 0)),
        out_shape=jax.ShapeDtypeStruct((RT, N, D), jnp.float32),
    )(xb, Wall)


def _einsum2_body(p0_ref, p1_ref, xw_ref, b_ref, w_ref, o_ref):
    h = p0_ref[0] + p1_ref[0] + xw_ref[0] + b_ref[0]
    hb = jnp.maximum(h, 0.0).astype(jnp.bfloat16)
    for r in range(RT):
        o_ref[r] = jnp.dot(hb, w_ref[r], preferred_element_type=jnp.float32)


def _tc_einsum2(parts, xw1, b1, Wall):
    return pl.pallas_call(
        _einsum2_body,
        grid=(N // _BN,),
        in_specs=[
            pl.BlockSpec((1, _BN, D), lambda i: (0, i, 0)),
            pl.BlockSpec((1, _BN, D), lambda i: (1, i, 0)),
            pl.BlockSpec((1, _BN, D), lambda i: (R, i, 0)),
            pl.BlockSpec((1, D), lambda i: (0, 0)),
            pl.BlockSpec((RT, D, D), lambda i: (0, 0, 0)),
        ],
        out_specs=pl.BlockSpec((RT, _BN, D), lambda i: (0, i, 0)),
        out_shape=jax.ShapeDtypeStruct((RT, N, D), jnp.float32),
    )(parts, parts, xw1, b1, Wall)


def _combine_body(p0_ref, p1_ref, xw_ref, b_ref, o_ref):
    o_ref[...] = p0_ref[0] + p1_ref[0] + xw_ref[0] + b_ref[0]


def _tc_combine(parts, xw2, b2):
    return pl.pallas_call(
        _combine_body,
        grid=(N // _BN,),
        in_specs=[
            pl.BlockSpec((1, _BN, D), lambda i: (0, i, 0)),
            pl.BlockSpec((1, _BN, D), lambda i: (1, i, 0)),
            pl.BlockSpec((1, _BN, D), lambda i: (R, i, 0)),
            pl.BlockSpec((1, D), lambda i: (0, 0)),
        ],
        out_specs=pl.BlockSpec((_BN, D), lambda i: (i, 0)),
        out_shape=jax.ShapeDtypeStruct((N, D), jnp.float32),
    )(parts, parts, xw2, b2)


def kernel(x, edge_index, edge_type, W1, Wroot1, b1, W2, Wroot2, b2):
    rel = edge_type
    Wall1 = jnp.concatenate([W1, Wroot1[None]], axis=0).astype(jnp.bfloat16)
    Wall2 = jnp.concatenate([W2, Wroot2[None]], axis=0).astype(jnp.bfloat16)
    xb = x.astype(jnp.bfloat16)
    b1r = b1.reshape(1, D)
    b2r = b2.reshape(1, D)
    zeros_nr = jnp.zeros((N * R,), jnp.float32)
    zeros_nd = jnp.zeros((N, D), jnp.float32)

    gidx, norm = _sc_prep(edge_index, rel, zeros_nr)

    xw1 = _tc_einsum1(xb, Wall1)
    parts1 = _sc_agg(xw1, gidx, edge_index, norm, zeros_nd)
    xw2 = _tc_einsum2(parts1, xw1, b1r, Wall2)
    parts2 = _sc_agg(xw2, gidx, edge_index, norm, zeros_nd)
    out = _tc_combine(parts2, xw2, b2r)
    return out


# speedup vs baseline: 1.1679x; 1.0113x over previous
import functools

import jax
import jax.numpy as jnp
from jax import lax
from jax.experimental import pallas as pl
from jax.experimental.pallas import tpu as pltpu
from jax.experimental.pallas import tpu_sc as plsc

N = 10000
E = 320000
R = 8
D = 128
RT = R + 1

NC = 2
NS = 16
NW = NC * NS
EPT = E // NW
CH = 80
NCHUNK = EPT // CH
NPT = N // NS
CPB = 25
NBLK = EPT // (CPB * CH)
NROW = NW * NBLK
ACH = 100
ACPB = 20
ABLK = EPT // (ACPB * ACH)
AROW = NW * ABLK

_mesh = functools.partial(
    plsc.VectorSubcoreMesh, core_axis_name="c", subcore_axis_name="s",
    num_cores=NC, num_subcores=NS)

_sc_params = pltpu.CompilerParams(needs_layout_passes=False)

_L = 16


EPS = E // NS
BPS = NBLK * NC


def _sc_prep_body(ei_h, rel_h, zeros_h,
                  gidx_h, norm_h,
                  src_v, dst_v, rel_v, gidx_v, comb_v, ones_v, norm_v,
                  cnt_v, sem, cnt_sp):
    c = lax.axis_index("c")
    s = lax.axis_index("s")
    wid = c * NS + s
    coff = jnp.minimum(s * 5120, N * R - 5120)
    pltpu.sync_copy(zeros_h.at[pl.ds(coff, 5120)], cnt_sp.at[pl.ds(coff, 5120)])

    def fill_ones(j, _):
        ones_v[pl.ds(j * _L, _L)] = jnp.ones((_L,), jnp.float32)
        return 0

    lax.fori_loop(0, CH // _L, fill_ones, 0)
    plsc.subcore_barrier()

    def blockA(bk, _):
        base = s * BPS + bk
        pltpu.sync_copy(ei_h.at[0, base], src_v)
        pltpu.sync_copy(ei_h.at[1, base], dst_v)
        pltpu.sync_copy(rel_h.at[base], rel_v)

        def lane(i, _):
            for m in range(CH // _L):
                sl = pl.ds(m * _L, _L)
                rv = rel_v[i, sl]
                gidx_v[i, sl] = rv * N + src_v[i, sl]
                comb_v[i, sl] = dst_v[i, sl] * R + rv
            return 0

        lax.fori_loop(0, CPB, lane, 0)

        @pl.when(c == 0)
        def _():
            pltpu.sync_copy(gidx_v, gidx_h.at[base])

        for j in range(CPB):
            pltpu.async_copy(ones_v, cnt_sp.at[comb_v.at[j]], sem, add=True)
        for j in range(CPB):
            pltpu.make_async_copy(ones_v, cnt_sp.at[comb_v.at[j]], sem).wait()
        return 0

    lax.fori_loop(0, BPS, blockA, 0)
    plsc.subcore_barrier()
    pltpu.sync_copy(cnt_sp, cnt_v)

    def blockB(bk, _):
        base = wid * NBLK + bk
        pltpu.sync_copy(ei_h.at[1, base], dst_v)
        pltpu.sync_copy(rel_h.at[base], rel_v)

        def lane(i, _):
            for m in range(CH // _L):
                sl = pl.ds(m * _L, _L)
                cb = dst_v[i, sl] * R + rel_v[i, sl]
                cv = plsc.load_gather(cnt_v, [cb])
                norm_v[i, sl] = 1.0 / jnp.maximum(cv, 1.0)
            return 0

        lax.fori_loop(0, CPB, lane, 0)
        pltpu.sync_copy(norm_v, norm_h.at[base])
        return 0

    lax.fori_loop(0, NBLK, blockB, 0)


def _sc_prep(ei, rel, zeros_nr):
    k = functools.partial(
        pl.kernel,
        out_type=[jax.ShapeDtypeStruct((NROW, CPB, CH), jnp.int32),
                  jax.ShapeDtypeStruct((NROW, CPB, CH), jnp.float32)],
        mesh=_mesh(),
        compiler_params=_sc_params,
        scratch_types=[
            pltpu.VMEM((CPB, CH), jnp.int32),
            pltpu.VMEM((CPB, CH), jnp.int32),
            pltpu.VMEM((CPB, CH), jnp.int32),
            pltpu.VMEM((CPB, CH), jnp.int32),
            pltpu.VMEM((CPB, CH), jnp.int32),
            pltpu.VMEM((CH,), jnp.float32),
            pltpu.VMEM((CPB, CH), jnp.float32),
            pltpu.VMEM((N * R,), jnp.float32),
            pltpu.SemaphoreType.DMA,
            pltpu.VMEM_SHARED((N * R,), jnp.float32),
        ],
    )(_sc_prep_body)
    return k(ei.reshape(2, NROW, CPB, CH), rel.reshape(NROW, CPB, CH),
             zeros_nr)


def _sc_agg_body(xw_h, gidx_h, ei_h, norm_h, zeros_h, parts_h,
                 gidx_v, dst_v, norm_v, r0, r1, g0, g1, s0, s1, acc_sp):
    rows = (r0, r1)
    gsem = (g0, g1)
    ssem = (s0, s1)
    c = lax.axis_index("c")
    s = lax.axis_index("s")
    wid = c * NS + s
    roff = jnp.minimum(s * 640, N - 640)
    pltpu.sync_copy(zeros_h.at[pl.ds(roff, 640), :],
                    acc_sp.at[pl.ds(roff, 640), :])
    plsc.subcore_barrier()

    def issue_gather(t, b):
        pltpu.async_copy(xw_h.at[gidx_v.at[t]], rows[b], gsem[b])

    def wait_gather(t, b):
        pltpu.make_async_copy(xw_h.at[gidx_v.at[t]], rows[b], gsem[b]).wait()

    def issue_scatter(t, b):
        pltpu.async_copy(rows[b], acc_sp.at[dst_v.at[t]], ssem[b], add=True)

    def drain_scatter(b):
        pltpu.make_async_copy(rows[b], acc_sp.at[dst_v.at[0]], ssem[b]).wait()

    def block(bk, _):
        base = wid * ABLK + bk
        pltpu.sync_copy(gidx_h.at[base], gidx_v)
        pltpu.sync_copy(ei_h.at[1, base], dst_v)
        pltpu.sync_copy(norm_h.at[base], norm_v)
        issue_gather(0, 0)

        def group(g, _):
            for b in range(2):
                t = g * 2 + b
                wait_gather(t, b)
                o = 1 - b
                if b == 0:
                    @pl.when(g >= 1)
                    def _():
                        drain_scatter(o)
                    issue_gather(t + 1, o)
                else:
                    drain_scatter(o)

                    @pl.when(g < ACPB // 2 - 1)
                    def _():
                        issue_gather(t + 1, o)

                def row(j, _):
                    nv = plsc.load_gather(
                        norm_v, [jnp.full((_L,), t, jnp.int32),
                                 jnp.full((_L,), j, jnp.int32)])
                    for kk in range(D // _L):
                        sl = pl.ds(kk * _L, _L)
                        rows[b][j, sl] = rows[b][j, sl] * nv
                    return 0

                lax.fori_loop(0, ACH, row, 0)
                issue_scatter(t, b)
            return 0

        lax.fori_loop(0, ACPB // 2, group, 0)
        drain_scatter(1)
        return 0

    lax.fori_loop(0, ABLK, block, 0)
    plsc.subcore_barrier()
    pltpu.sync_copy(acc_sp.at[pl.ds(roff, 640), :],
                    parts_h.at[c, pl.ds(roff, 640), :])


def _sc_agg(xw, gidx, ei, norm, zeros_nd):
    k = functools.partial(
        pl.kernel,
        out_type=jax.ShapeDtypeStruct((NC, N, D), jnp.float32),
        mesh=_mesh(),
        compiler_params=_sc_params,
        scratch_types=(
            [pltpu.VMEM((ACPB, ACH), jnp.int32),
             pltpu.VMEM((ACPB, ACH), jnp.int32),
             pltpu.VMEM((ACPB, ACH), jnp.float32)]
            + [pltpu.VMEM((ACH, D), jnp.float32)] * 2
            + [pltpu.SemaphoreType.DMA] * 4
            + [pltpu.VMEM_SHARED((N, D), jnp.float32)]
        ),
    )(_sc_agg_body)
    return k(xw.reshape(N * RT, D), gidx.reshape(AROW, ACPB, ACH),
             ei.reshape(2, AROW, ACPB, ACH), norm.reshape(AROW, ACPB, ACH),
             zeros_nd)


_BN = 400


def _einsum1_body(x_ref, w_ref, o_ref):
    xb = x_ref[...]
    for r in range(RT):
        o_ref[r] = jnp.dot(xb, w_ref[r], preferred_element_type=jnp.float32)


def _tc_einsum1(xb, Wall):
    return pl.pallas_call(
        _einsum1_body,
        grid=(N // _BN,),
        in_specs=[
            pl.BlockSpec((_BN, D), lambda i: (i, 0)),
            pl.BlockSpec((RT, D, D), lambda i: (0, 0, 0)),
        ],
        out_specs=pl.BlockSpec((RT, _BN, D), lambda i: (0, i, 0)),
        out_shape=jax.ShapeDtypeStruct((RT, N, D), jnp.float32),
    )(xb, Wall)


def _einsum2_body(p0_ref, p1_ref, xw_ref, b_ref, w_ref, o_ref):
    h = p0_ref[0] + p1_ref[0] + xw_ref[0] + b_ref[0]
    hb = jnp.maximum(h, 0.0).astype(jnp.bfloat16)
    for r in range(RT):
        o_ref[r] = jnp.dot(hb, w_ref[r], preferred_element_type=jnp.float32)


def _tc_einsum2(parts, xw1, b1, Wall):
    return pl.pallas_call(
        _einsum2_body,
        grid=(N // _BN,),
        in_specs=[
            pl.BlockSpec((1, _BN, D), lambda i: (0, i, 0)),
            pl.BlockSpec((1, _BN, D), lambda i: (1, i, 0)),
            pl.BlockSpec((1, _BN, D), lambda i: (R, i, 0)),
            pl.BlockSpec((1, D), lambda i: (0, 0)),
            pl.BlockSpec((RT, D, D), lambda i: (0, 0, 0)),
        ],
        out_specs=pl.BlockSpec((RT, _BN, D), lambda i: (0, i, 0)),
        out_shape=jax.ShapeDtypeStruct((RT, N, D), jnp.float32),
    )(parts, parts, xw1, b1, Wall)


def _combine_body(p0_ref, p1_ref, xw_ref, b_ref, o_ref):
    o_ref[...] = p0_ref[0] + p1_ref[0] + xw_ref[0] + b_ref[0]


def _tc_combine(parts, xw2, b2):
    return pl.pallas_call(
        _combine_body,
        grid=(N // _BN,),
        in_specs=[
            pl.BlockSpec((1, _BN, D), lambda i: (0, i, 0)),
            pl.BlockSpec((1, _BN, D), lambda i: (1, i, 0)),
            pl.BlockSpec((1, _BN, D), lambda i: (R, i, 0)),
            pl.BlockSpec((1, D), lambda i: (0, 0)),
        ],
        out_specs=pl.BlockSpec((_BN, D), lambda i: (i, 0)),
        out_shape=jax.ShapeDtypeStruct((N, D), jnp.float32),
    )(parts, parts, xw2, b2)


def kernel(x, edge_index, edge_type, W1, Wroot1, b1, W2, Wroot2, b2):
    rel = edge_type
    Wall1 = jnp.concatenate([W1, Wroot1[None]], axis=0).astype(jnp.bfloat16)
    Wall2 = jnp.concatenate([W2, Wroot2[None]], axis=0).astype(jnp.bfloat16)
    xb = x.astype(jnp.bfloat16)
    b1r = b1.reshape(1, D)
    b2r = b2.reshape(1, D)
    zeros_nr = jnp.zeros((N * R,), jnp.float32)
    zeros_nd = jnp.zeros((N, D), jnp.float32)

    gidx, norm = _sc_prep(edge_index, rel, zeros_nr)

    xw1 = _tc_einsum1(xb, Wall1)
    parts1 = _sc_agg(xw1, gidx, edge_index, norm, zeros_nd)
    xw2 = _tc_einsum2(parts1, xw1, b1r, Wall2)
    parts2 = _sc_agg(xw2, gidx, edge_index, norm, zeros_nd)
    out = _tc_combine(parts2, xw2, b2r)
    return out


# speedup vs baseline: 1.1711x; 1.0028x over previous
import functools

import jax
import jax.numpy as jnp
from jax import lax
from jax.experimental import pallas as pl
from jax.experimental.pallas import tpu as pltpu
from jax.experimental.pallas import tpu_sc as plsc

N = 10000
E = 320000
R = 8
D = 128
RT = R + 1

NC = 2
NS = 16
NW = NC * NS
EPT = E // NW
CH = 80
NCHUNK = EPT // CH
NPT = N // NS
CPB = 25
NBLK = EPT // (CPB * CH)
NROW = NW * NBLK
ACH = 125
ACPB = 16
ABLK = EPT // (ACPB * ACH)
AROW = NW * ABLK

_mesh = functools.partial(
    plsc.VectorSubcoreMesh, core_axis_name="c", subcore_axis_name="s",
    num_cores=NC, num_subcores=NS)

_sc_params = pltpu.CompilerParams(needs_layout_passes=False)

_L = 16


EPS = E // NS
BPS = NBLK * NC


def _sc_prep_body(ei_h, rel_h, zeros_h,
                  gidx_h, norm_h,
                  src_v, dst_v, rel_v, gidx_v, comb_v, ones_v, norm_v,
                  cnt_v, sem, cnt_sp):
    c = lax.axis_index("c")
    s = lax.axis_index("s")
    wid = c * NS + s
    coff = jnp.minimum(s * 5120, N * R - 5120)
    pltpu.sync_copy(zeros_h.at[pl.ds(coff, 5120)], cnt_sp.at[pl.ds(coff, 5120)])

    def fill_ones(j, _):
        ones_v[pl.ds(j * _L, _L)] = jnp.ones((_L,), jnp.float32)
        return 0

    lax.fori_loop(0, CH // _L, fill_ones, 0)
    plsc.subcore_barrier()

    def blockA(bk, _):
        base = s * BPS + bk
        pltpu.sync_copy(ei_h.at[0, base], src_v)
        pltpu.sync_copy(ei_h.at[1, base], dst_v)
        pltpu.sync_copy(rel_h.at[base], rel_v)

        def lane(i, _):
            for m in range(CH // _L):
                sl = pl.ds(m * _L, _L)
                rv = rel_v[i, sl]
                gidx_v[i, sl] = rv * N + src_v[i, sl]
                comb_v[i, sl] = dst_v[i, sl] * R + rv
            return 0

        lax.fori_loop(0, CPB, lane, 0)

        @pl.when(c == 0)
        def _():
            pltpu.sync_copy(gidx_v, gidx_h.at[base])

        for j in range(CPB):
            pltpu.async_copy(ones_v, cnt_sp.at[comb_v.at[j]], sem, add=True)
        for j in range(CPB):
            pltpu.make_async_copy(ones_v, cnt_sp.at[comb_v.at[j]], sem).wait()
        return 0

    lax.fori_loop(0, BPS, blockA, 0)
    plsc.subcore_barrier()
    pltpu.sync_copy(cnt_sp, cnt_v)

    def blockB(bk, _):
        base = wid * NBLK + bk
        pltpu.sync_copy(ei_h.at[1, base], dst_v)
        pltpu.sync_copy(rel_h.at[base], rel_v)

        def lane(i, _):
            for m in range(CH // _L):
                sl = pl.ds(m * _L, _L)
                cb = dst_v[i, sl] * R + rel_v[i, sl]
                cv = plsc.load_gather(cnt_v, [cb])
                norm_v[i, sl] = 1.0 / jnp.maximum(cv, 1.0)
            return 0

        lax.fori_loop(0, CPB, lane, 0)
        pltpu.sync_copy(norm_v, norm_h.at[base])
        return 0

    lax.fori_loop(0, NBLK, blockB, 0)


def _sc_prep(ei, rel, zeros_nr):
    k = functools.partial(
        pl.kernel,
        out_type=[jax.ShapeDtypeStruct((NROW, CPB, CH), jnp.int32),
                  jax.ShapeDtypeStruct((NROW, CPB, CH), jnp.float32)],
        mesh=_mesh(),
        compiler_params=_sc_params,
        scratch_types=[
            pltpu.VMEM((CPB, CH), jnp.int32),
            pltpu.VMEM((CPB, CH), jnp.int32),
            pltpu.VMEM((CPB, CH), jnp.int32),
            pltpu.VMEM((CPB, CH), jnp.int32),
            pltpu.VMEM((CPB, CH), jnp.int32),
            pltpu.VMEM((CH,), jnp.float32),
            pltpu.VMEM((CPB, CH), jnp.float32),
            pltpu.VMEM((N * R,), jnp.float32),
            pltpu.SemaphoreType.DMA,
            pltpu.VMEM_SHARED((N * R,), jnp.float32),
        ],
    )(_sc_prep_body)
    return k(ei.reshape(2, NROW, CPB, CH), rel.reshape(NROW, CPB, CH),
             zeros_nr)


def _sc_agg_body(xw_h, gidx_h, ei_h, norm_h, zeros_h, parts_h,
                 gidx_v, dst_v, norm_v, r0, r1, g0, g1, s0, s1, acc_sp):
    rows = (r0, r1)
    gsem = (g0, g1)
    ssem = (s0, s1)
    c = lax.axis_index("c")
    s = lax.axis_index("s")
    wid = c * NS + s
    roff = jnp.minimum(s * 640, N - 640)
    pltpu.sync_copy(zeros_h.at[pl.ds(roff, 640), :],
                    acc_sp.at[pl.ds(roff, 640), :])
    plsc.subcore_barrier()

    def issue_gather(t, b):
        pltpu.async_copy(xw_h.at[gidx_v.at[t]], rows[b], gsem[b])

    def wait_gather(t, b):
        pltpu.make_async_copy(xw_h.at[gidx_v.at[t]], rows[b], gsem[b]).wait()

    def issue_scatter(t, b):
        pltpu.async_copy(rows[b], acc_sp.at[dst_v.at[t]], ssem[b], add=True)

    def drain_scatter(b):
        pltpu.make_async_copy(rows[b], acc_sp.at[dst_v.at[0]], ssem[b]).wait()

    def block(bk, _):
        base = wid * ABLK + bk
        pltpu.sync_copy(gidx_h.at[base], gidx_v)
        pltpu.sync_copy(ei_h.at[1, base], dst_v)
        pltpu.sync_copy(norm_h.at[base], norm_v)
        issue_gather(0, 0)

        def group(g, _):
            for b in range(2):
                t = g * 2 + b
                wait_gather(t, b)
                o = 1 - b
                if b == 0:
                    @pl.when(g >= 1)
                    def _():
                        drain_scatter(o)
                    issue_gather(t + 1, o)
                else:
                    drain_scatter(o)

                    @pl.when(g < ACPB // 2 - 1)
                    def _():
                        issue_gather(t + 1, o)

                def row(j, _):
                    nv = plsc.load_gather(
                        norm_v, [jnp.full((_L,), t, jnp.int32),
                                 jnp.full((_L,), j, jnp.int32)])
                    for kk in range(D // _L):
                        sl = pl.ds(kk * _L, _L)
                        rows[b][j, sl] = rows[b][j, sl] * nv
                    return 0

                lax.fori_loop(0, ACH, row, 0)
                issue_scatter(t, b)
            return 0

        lax.fori_loop(0, ACPB // 2, group, 0)
        drain_scatter(1)
        return 0

    lax.fori_loop(0, ABLK, block, 0)
    plsc.subcore_barrier()
    pltpu.sync_copy(acc_sp.at[pl.ds(roff, 640), :],
                    parts_h.at[c, pl.ds(roff, 640), :])


def _sc_agg(xw, gidx, ei, norm, zeros_nd):
    k = functools.partial(
        pl.kernel,
        out_type=jax.ShapeDtypeStruct((NC, N, D), jnp.float32),
        mesh=_mesh(),
        compiler_params=_sc_params,
        scratch_types=(
            [pltpu.VMEM((ACPB, ACH), jnp.int32),
             pltpu.VMEM((ACPB, ACH), jnp.int32),
             pltpu.VMEM((ACPB, ACH), jnp.float32)]
            + [pltpu.VMEM((ACH, D), jnp.float32)] * 2
            + [pltpu.SemaphoreType.DMA] * 4
            + [pltpu.VMEM_SHARED((N, D), jnp.float32)]
        ),
    )(_sc_agg_body)
    return k(xw.reshape(N * RT, D), gidx.reshape(AROW, ACPB, ACH),
             ei.reshape(2, AROW, ACPB, ACH), norm.reshape(AROW, ACPB, ACH),
             zeros_nd)


_BN = 400


def _einsum1_body(x_ref, w_ref, o_ref):
    xb = x_ref[...]
    for r in range(RT):
        o_ref[r] = jnp.dot(xb, w_ref[r], preferred_element_type=jnp.float32)


def _tc_einsum1(xb, Wall):
    return pl.pallas_call(
        _einsum1_body,
        grid=(N // _BN,),
        in_specs=[
            pl.BlockSpec((_BN, D), lambda i: (i, 0)),
            pl.BlockSpec((RT, D, D), lambda i: (0, 0, 0)),
        ],
        out_specs=pl.BlockSpec((RT, _BN, D), lambda i: (0, i, 0)),
        out_shape=jax.ShapeDtypeStruct((RT, N, D), jnp.float32),
    )(xb, Wall)


def _einsum2_body(p0_ref, p1_ref, xw_ref, b_ref, w_ref, o_ref):
    h = p0_ref[0] + p1_ref[0] + xw_ref[0] + b_ref[0]
    hb = jnp.maximum(h, 0.0).astype(jnp.bfloat16)
    for r in range(RT):
        o_ref[r] = jnp.dot(hb, w_ref[r], preferred_element_type=jnp.float32)


def _tc_einsum2(parts, xw1, b1, Wall):
    return pl.pallas_call(
        _einsum2_body,
        grid=(N // _BN,),
        in_specs=[
            pl.BlockSpec((1, _BN, D), lambda i: (0, i, 0)),
            pl.BlockSpec((1, _BN, D), lambda i: (1, i, 0)),
            pl.BlockSpec((1, _BN, D), lambda i: (R, i, 0)),
            pl.BlockSpec((1, D), lambda i: (0, 0)),
            pl.BlockSpec((RT, D, D), lambda i: (0, 0, 0)),
        ],
        out_specs=pl.BlockSpec((RT, _BN, D), lambda i: (0, i, 0)),
        out_shape=jax.ShapeDtypeStruct((RT, N, D), jnp.float32),
    )(parts, parts, xw1, b1, Wall)


def _combine_body(p0_ref, p1_ref, xw_ref, b_ref, o_ref):
    o_ref[...] = p0_ref[0] + p1_ref[0] + xw_ref[0] + b_ref[0]


def _tc_combine(parts, xw2, b2):
    return pl.pallas_call(
        _combine_body,
        grid=(N // _BN,),
        in_specs=[
            pl.BlockSpec((1, _BN, D), lambda i: (0, i, 0)),
            pl.BlockSpec((1, _BN, D), lambda i: (1, i, 0)),
            pl.BlockSpec((1, _BN, D), lambda i: (R, i, 0)),
            pl.BlockSpec((1, D), lambda i: (0, 0)),
        ],
        out_specs=pl.BlockSpec((_BN, D), lambda i: (i, 0)),
        out_shape=jax.ShapeDtypeStruct((N, D), jnp.float32),
    )(parts, parts, xw2, b2)


def kernel(x, edge_index, edge_type, W1, Wroot1, b1, W2, Wroot2, b2):
    rel = edge_type
    Wall1 = jnp.concatenate([W1, Wroot1[None]], axis=0).astype(jnp.bfloat16)
    Wall2 = jnp.concatenate([W2, Wroot2[None]], axis=0).astype(jnp.bfloat16)
    xb = x.astype(jnp.bfloat16)
    b1r = b1.reshape(1, D)
    b2r = b2.reshape(1, D)
    zeros_nr = jnp.zeros((N * R,), jnp.float32)
    zeros_nd = jnp.zeros((N, D), jnp.float32)

    gidx, norm = _sc_prep(edge_index, rel, zeros_nr)

    xw1 = _tc_einsum1(xb, Wall1)
    parts1 = _sc_agg(xw1, gidx, edge_index, norm, zeros_nd)
    xw2 = _tc_einsum2(parts1, xw1, b1r, Wall2)
    parts2 = _sc_agg(xw2, gidx, edge_index, norm, zeros_nd)
    out = _tc_combine(parts2, xw2, b2r)
    return out
